# baseline (speedup 1.0000x reference)
import functools
import math

import jax
import jax.numpy as jnp
from jax import lax
from jax.experimental import pallas as pl
from jax.experimental.pallas import tpu as pltpu
from jax.experimental.pallas import tpu_sc as plsc

D = 64
S = 200
LANES = 16
NC, NS = 2, 16
NW = NC * NS
NB = 128
NG = NB // LANES

SCALE = math.sqrt(D)


def _emb_body(B, x_hbm, tbl_hbm, pe_hbm, out_hbm,
              idxr0, idxr1, idxh0, idxh1, gat0, gat1, ob0, ob1, pe_v,
              semg0, semg1, sems0, sems1, semi0, semi1):
    wid = lax.axis_index("s") * NC + lax.axis_index("c")
    b0 = wid * NB

    idxr = (idxr0, idxr1)
    idxh = (idxh0, idxh1)
    gat = (gat0, gat1)
    ob = (ob0, ob1)
    semg = (semg0, semg1)
    sems = (sems0, sems1)
    semi = (semi0, semi1)

    lane = lax.iota(jnp.int32, LANES)

    def prep_and_fire(p, s):
        for g in range(NG):
            v = idxr[p][pl.ds(g * LANES, LANES)]
            idxh[p][pl.ds(g * LANES, LANES)] = lax.shift_right_logical(v, 1)
        pltpu.async_copy(tbl_hbm.at[idxh[p]], gat[p], semg[p])

    def compute(p, s):
        rows = []
        hoffs = []
        for g in range(NG):
            v = idxr[p][pl.ds(g * LANES, LANES)]
            rows.append(lane + (g * LANES))
            hoffs.append(lax.shift_left(lax.bitwise_and(v, 1), 6))

        def dbody(d, carry):
            rws, hfs = carry
            pidx = lax.broadcast(s * D + d, (LANES,))
            pv = plsc.load_gather(pe_v, [pidx])
            for g in range(NG):
                col = hfs[g] + d
                val = plsc.load_gather(gat[p], [rws[g], col])
                ob[p][d, pl.ds(g * LANES, LANES)] = val * SCALE + pv
            return (rws, hfs)

        lax.fori_loop(0, D, dbody, (tuple(rows), tuple(hoffs)))

    pltpu.sync_copy(pe_hbm, pe_v)

    pltpu.sync_copy(x_hbm.at[pl.ds(b0, NB)], idxr[0])
    prep_and_fire(0, 0)
    pltpu.async_copy(x_hbm.at[pl.ds(B + b0, NB)], idxr[1], semi[1])

    def body_for(b, s):
        o = 1 - b

        @pl.when(s >= 1)
        def _():
            pltpu.make_async_copy(
                ob[o], out_hbm.at[s - 1, :, pl.ds(b0, NB)], sems[o]).wait()

        @pl.when(s + 1 < S)
        def _():
            pltpu.make_async_copy(
                x_hbm.at[pl.ds((s + 1) * B + b0, NB)], idxr[o],
                semi[o]).wait()
            prep_and_fire(o, s + 1)

        pltpu.make_async_copy(
            tbl_hbm.at[pl.ds(0, NB)], gat[b], semg[b]).wait()

        compute(b, s)

        @pl.when(s + 2 < S)
        def _():
            pltpu.async_copy(
                x_hbm.at[pl.ds((s + 2) * B + b0, NB)], idxr[b], semi[b])

        pltpu.async_copy(ob[b], out_hbm.at[s, :, pl.ds(b0, NB)], sems[b])

    def outer(i, carry):
        body_for(0, i * 2)
        body_for(1, i * 2 + 1)
        return carry

    lax.fori_loop(0, S // 2, outer, 0)

    pltpu.make_async_copy(
        ob[1], out_hbm.at[S - 1, :, pl.ds(b0, NB)], sems[1]).wait()


def kernel(x, table, pe):
    B, seq = x.shape
    x_lin = x.T.astype(jnp.int32).reshape(seq * B)
    tbl2 = table.reshape(table.shape[0] // 2, 2 * D)
    pe_lin = pe[:seq].reshape(seq * D)

    mesh = plsc.VectorSubcoreMesh(core_axis_name="c", subcore_axis_name="s")
    run = pl.kernel(
        functools.partial(_emb_body, B),
        out_type=jax.ShapeDtypeStruct((seq, D, B), jnp.float32),
        mesh=mesh,
        scratch_types=[
            pltpu.VMEM((NB,), jnp.int32),
            pltpu.VMEM((NB,), jnp.int32),
            pltpu.VMEM((NB,), jnp.int32),
            pltpu.VMEM((NB,), jnp.int32),
            pltpu.VMEM((NB, 2 * D), jnp.float32),
            pltpu.VMEM((NB, 2 * D), jnp.float32),
            pltpu.VMEM((D, NB), jnp.float32),
            pltpu.VMEM((D, NB), jnp.float32),
            pltpu.VMEM((S * D,), jnp.float32),
            pltpu.SemaphoreType.DMA,
            pltpu.SemaphoreType.DMA,
            pltpu.SemaphoreType.DMA,
            pltpu.SemaphoreType.DMA,
            pltpu.SemaphoreType.DMA,
            pltpu.SemaphoreType.DMA,
        ],
        compiler_params=pltpu.CompilerParams(
            use_tc_tiling_on_sc=True, needs_layout_passes=False),
    )
    out_t = run(x_lin, tbl2, pe_lin)
    return out_t.transpose(2, 0, 1)


# speedup vs baseline: 1.0012x; 1.0012x over previous
import functools
import math

import jax
import jax.numpy as jnp
from jax import lax
from jax.experimental import pallas as pl
from jax.experimental.pallas import tpu as pltpu
from jax.experimental.pallas import tpu_sc as plsc

D = 64
S = 200
LANES = 16
NC, NS = 2, 16
NW = NC * NS
NB = 128
NG = NB // LANES

SCALE = math.sqrt(D)


def _emb_body(B, x_hbm, tbl_hbm, pe_hbm, out_hbm,
              idxr0, idxr1, idxh0, idxh1, gat0, gat1, ob0, ob1, pe_v,
              semg0, semg1, sems0, sems1, semi0, semi1):
    wid = lax.axis_index("s") * NC + lax.axis_index("c")
    b0 = wid * NB

    idxr = (idxr0, idxr1)
    idxh = (idxh0, idxh1)
    gat = (gat0, gat1)
    ob = (ob0, ob1)
    semg = (semg0, semg1)
    sems = (sems0, sems1)
    semi = (semi0, semi1)

    lane = lax.iota(jnp.int32, LANES)

    def prep_and_fire(p, s):
        for g in range(NG):
            v = idxr[p][pl.ds(g * LANES, LANES)]
            idxh[p][pl.ds(g * LANES, LANES)] = lax.shift_right_logical(v, 1)
        pltpu.async_copy(tbl_hbm.at[idxh[p]], gat[p], semg[p])

    DU = 8

    def compute(p, s):
        rows = []
        hoffs = []
        for g in range(NG):
            v = idxr[p][pl.ds(g * LANES, LANES)]
            rows.append(lane + (g * LANES))
            hoffs.append(lax.shift_left(lax.bitwise_and(v, 1), 6))

        def dblock(i, carry):
            rws, hfs = carry
            d0 = i * DU
            for dd in range(DU):
                d = d0 + dd
                pidx = lax.broadcast(s * D + d, (LANES,))
                pv = plsc.load_gather(pe_v, [pidx])
                for g in range(NG):
                    col = hfs[g] + d
                    val = plsc.load_gather(gat[p], [rws[g], col])
                    ob[p][d, pl.ds(g * LANES, LANES)] = val * SCALE + pv
            return (rws, hfs)

        lax.fori_loop(0, D // DU, dblock, (tuple(rows), tuple(hoffs)))

    pltpu.sync_copy(pe_hbm, pe_v)

    pltpu.sync_copy(x_hbm.at[0, pl.ds(b0, NB)], idxr[0])
    prep_and_fire(0, 0)
    pltpu.async_copy(x_hbm.at[1, pl.ds(b0, NB)], idxr[1], semi[1])

    def body_for(b, s):
        o = 1 - b

        @pl.when(s >= 1)
        def _():
            pltpu.make_async_copy(
                ob[o], out_hbm.at[s - 1, :, pl.ds(b0, NB)], sems[o]).wait()

        @pl.when(s + 1 < S)
        def _():
            pltpu.make_async_copy(
                x_hbm.at[s + 1, pl.ds(b0, NB)], idxr[o], semi[o]).wait()
            prep_and_fire(o, s + 1)

        pltpu.make_async_copy(
            tbl_hbm.at[pl.ds(0, NB)], gat[b], semg[b]).wait()

        compute(b, s)

        @pl.when(s + 2 < S)
        def _():
            pltpu.async_copy(
                x_hbm.at[s + 2, pl.ds(b0, NB)], idxr[b], semi[b])

        pltpu.async_copy(ob[b], out_hbm.at[s, :, pl.ds(b0, NB)], sems[b])

    def outer(i, carry):
        body_for(0, i * 2)
        body_for(1, i * 2 + 1)
        return carry

    lax.fori_loop(0, S // 2, outer, 0)

    pltpu.make_async_copy(
        ob[1], out_hbm.at[S - 1, :, pl.ds(b0, NB)], sems[1]).wait()


def kernel(x, table, pe):
    B, seq = x.shape
    x_lin = x.T.astype(jnp.int32)
    tbl2 = table.reshape(table.shape[0] // 2, 2 * D)
    pe_lin = pe[:seq].reshape(seq * D)

    mesh = plsc.VectorSubcoreMesh(core_axis_name="c", subcore_axis_name="s")
    run = pl.kernel(
        functools.partial(_emb_body, B),
        out_type=jax.ShapeDtypeStruct((seq, D, B), jnp.float32),
        mesh=mesh,
        scratch_types=[
            pltpu.VMEM((NB,), jnp.int32),
            pltpu.VMEM((NB,), jnp.int32),
            pltpu.VMEM((NB,), jnp.int32),
            pltpu.VMEM((NB,), jnp.int32),
            pltpu.VMEM((NB, 2 * D), jnp.float32),
            pltpu.VMEM((NB, 2 * D), jnp.float32),
            pltpu.VMEM((D, NB), jnp.float32),
            pltpu.VMEM((D, NB), jnp.float32),
            pltpu.VMEM((S * D,), jnp.float32),
            pltpu.SemaphoreType.DMA,
            pltpu.SemaphoreType.DMA,
            pltpu.SemaphoreType.DMA,
            pltpu.SemaphoreType.DMA,
            pltpu.SemaphoreType.DMA,
            pltpu.SemaphoreType.DMA,
        ],
        compiler_params=pltpu.CompilerParams(
            use_tc_tiling_on_sc=True, needs_layout_passes=False),
    )
    out_t = run(x_lin, tbl2, pe_lin)
    return out_t.transpose(2, 0, 1)


# speedup vs baseline: 1.5724x; 1.5705x over previous
import functools
import math

import jax
import jax.numpy as jnp
from jax import lax
from jax.experimental import pallas as pl
from jax.experimental.pallas import tpu as pltpu
from jax.experimental.pallas import tpu_sc as plsc

D = 64
S = 200
LANES = 16
NC, NS = 2, 16
NW = NC * NS
NB = 128
NG = NB // LANES

SCALE = math.sqrt(D)


def _emb_body(B, x_hbm, tbl_hbm, pe_hbm, out_hbm,
              idxr0, idxr1, idxh0, idxh1, gat0, gat1, ob0, ob1, pe_v,
              semg0, semg1, sems0, sems1, semi0, semi1):
    wid = lax.axis_index("s") * NC + lax.axis_index("c")
    b0 = wid * NB

    idxr = (idxr0, idxr1)
    idxh = (idxh0, idxh1)
    gat = (gat0, gat1)
    ob = (ob0, ob1)
    semg = (semg0, semg1)
    sems = (sems0, sems1)
    semi = (semi0, semi1)

    lane = lax.iota(jnp.int32, LANES)

    def prep_and_fire(p, s):
        for g in range(NG):
            v = idxr[p][pl.ds(g * LANES, LANES)]
            idxh[p][pl.ds(g * LANES, LANES)] = lax.shift_right_logical(v, 1)
        pltpu.async_copy(tbl_hbm.at[idxh[p]], gat[p], semg[p])

    def compute(p, s):
        rows = []
        hoffs = []
        for g in range(NG):
            v = idxr[p][pl.ds(g * LANES, LANES)]
            rows.append(lane + (g * LANES))
            hoffs.append(lax.shift_left(lax.bitwise_and(v, 1), 6))

        def dbody(d):
            pidx = lax.broadcast(s * D + d, (LANES,))
            pv = plsc.load_gather(pe_v, [pidx])
            for g in range(NG):
                col = hoffs[g] + d
                val = plsc.load_gather(gat[p], [rows[g], col])
                ob[p][d, pl.ds(g * LANES, LANES)] = val * SCALE + pv

        plsc.parallel_loop(0, D, 1, unroll=8)(dbody)

    pltpu.sync_copy(pe_hbm, pe_v)

    pltpu.sync_copy(x_hbm.at[0, pl.ds(b0, NB)], idxr[0])
    prep_and_fire(0, 0)
    pltpu.async_copy(x_hbm.at[1, pl.ds(b0, NB)], idxr[1], semi[1])

    def body_for(b, s):
        o = 1 - b

        @pl.when(s >= 1)
        def _():
            pltpu.make_async_copy(
                ob[o], out_hbm.at[s - 1, :, pl.ds(b0, NB)], sems[o]).wait()

        @pl.when(s + 1 < S)
        def _():
            pltpu.make_async_copy(
                x_hbm.at[s + 1, pl.ds(b0, NB)], idxr[o], semi[o]).wait()
            prep_and_fire(o, s + 1)

        pltpu.make_async_copy(
            tbl_hbm.at[pl.ds(0, NB)], gat[b], semg[b]).wait()

        compute(b, s)

        @pl.when(s + 2 < S)
        def _():
            pltpu.async_copy(
                x_hbm.at[s + 2, pl.ds(b0, NB)], idxr[b], semi[b])

        pltpu.async_copy(ob[b], out_hbm.at[s, :, pl.ds(b0, NB)], sems[b])

    def outer(i, carry):
        body_for(0, i * 2)
        body_for(1, i * 2 + 1)
        return carry

    lax.fori_loop(0, S // 2, outer, 0)

    pltpu.make_async_copy(
        ob[1], out_hbm.at[S - 1, :, pl.ds(b0, NB)], sems[1]).wait()


def kernel(x, table, pe):
    B, seq = x.shape
    x_lin = x.T.astype(jnp.int32)
    tbl2 = table.reshape(table.shape[0] // 2, 2 * D)
    pe_lin = pe[:seq].reshape(seq * D)

    mesh = plsc.VectorSubcoreMesh(core_axis_name="c", subcore_axis_name="s")
    run = pl.kernel(
        functools.partial(_emb_body, B),
        out_type=jax.ShapeDtypeStruct((seq, D, B), jnp.float32),
        mesh=mesh,
        scratch_types=[
            pltpu.VMEM((NB,), jnp.int32),
            pltpu.VMEM((NB,), jnp.int32),
            pltpu.VMEM((NB,), jnp.int32),
            pltpu.VMEM((NB,), jnp.int32),
            pltpu.VMEM((NB, 2 * D), jnp.float32),
            pltpu.VMEM((NB, 2 * D), jnp.float32),
            pltpu.VMEM((D, NB), jnp.float32),
            pltpu.VMEM((D, NB), jnp.float32),
            pltpu.VMEM((S * D,), jnp.float32),
            pltpu.SemaphoreType.DMA,
            pltpu.SemaphoreType.DMA,
            pltpu.SemaphoreType.DMA,
            pltpu.SemaphoreType.DMA,
            pltpu.SemaphoreType.DMA,
            pltpu.SemaphoreType.DMA,
        ],
        compiler_params=pltpu.CompilerParams(
            use_tc_tiling_on_sc=True, needs_layout_passes=False),
    )
    out_t = run(x_lin, tbl2, pe_lin)
    return out_t.transpose(2, 0, 1)


# speedup vs baseline: 1.5733x; 1.0005x over previous
import functools
import math

import jax
import jax.numpy as jnp
from jax import lax
from jax.experimental import pallas as pl
from jax.experimental.pallas import tpu as pltpu
from jax.experimental.pallas import tpu_sc as plsc

D = 64
S = 200
LANES = 16
NC, NS = 2, 16
NW = NC * NS
NB = 128
NG = NB // LANES
RING = 4

SCALE = math.sqrt(D)


def _emb_body(B, x_hbm, tbl_hbm, pe_hbm, out_hbm,
              idxr, idxh, gat, ob, pe_v, semg, sems, semi):
    wid = lax.axis_index("s") * NC + lax.axis_index("c")
    b0 = wid * NB

    lane = lax.iota(jnp.int32, LANES)

    def prep_and_fire(q):
        for g in range(NG):
            v = idxr[q][pl.ds(g * LANES, LANES)]
            idxh[q][pl.ds(g * LANES, LANES)] = lax.shift_right_logical(v, 1)
        pltpu.async_copy(tbl_hbm.at[idxh[q]], gat[q], semg[q])

    def compute(p, s):
        rows = []
        hoffs = []
        for g in range(NG):
            v = idxr[p][pl.ds(g * LANES, LANES)]
            rows.append(lane + (g * LANES))
            hoffs.append(lax.shift_left(lax.bitwise_and(v, 1), 6))

        def dbody(d):
            pidx = lax.broadcast(s * D + d, (LANES,))
            pv = plsc.load_gather(pe_v, [pidx])
            for g in range(NG):
                col = hoffs[g] + d
                val = plsc.load_gather(gat[p], [rows[g], col])
                ob[p][d, pl.ds(g * LANES, LANES)] = val * SCALE + pv

        plsc.parallel_loop(0, D, 1, unroll=8)(dbody)

    pltpu.sync_copy(pe_hbm, pe_v)

    pltpu.sync_copy(x_hbm.at[0, pl.ds(b0, NB)], idxr[0])
    prep_and_fire(0)
    pltpu.sync_copy(x_hbm.at[1, pl.ds(b0, NB)], idxr[1])
    prep_and_fire(1)
    pltpu.async_copy(x_hbm.at[2, pl.ds(b0, NB)], idxr[2], semi[2])

    def body_for(b, s):
        q = (b + 2) % RING
        r = (b + 3) % RING

        @pl.when(s + 2 < S)
        def _():
            @pl.when(s >= 2)
            def _():
                pltpu.make_async_copy(
                    ob[q], out_hbm.at[s - 2, :, pl.ds(b0, NB)],
                    sems[q]).wait()
            pltpu.make_async_copy(
                x_hbm.at[s + 2, pl.ds(b0, NB)], idxr[q], semi[q]).wait()
            prep_and_fire(q)

        pltpu.make_async_copy(
            tbl_hbm.at[pl.ds(0, NB)], gat[b], semg[b]).wait()

        compute(b, s)

        @pl.when(s + 3 < S)
        def _():
            pltpu.async_copy(
                x_hbm.at[s + 3, pl.ds(b0, NB)], idxr[r], semi[r])

        pltpu.async_copy(ob[b], out_hbm.at[s, :, pl.ds(b0, NB)], sems[b])

    def outer(i, carry):
        for b in range(RING):
            body_for(b, i * RING + b)
        return carry

    lax.fori_loop(0, S // RING, outer, 0)

    for sf in range(S - RING, S):
        pltpu.make_async_copy(
            ob[sf % RING], out_hbm.at[sf, :, pl.ds(b0, NB)],
            sems[sf % RING]).wait()


def _body_adapter(B, x_hbm, tbl_hbm, pe_hbm, out_hbm, *scratch):
    idxr = scratch[0:RING]
    idxh = scratch[RING:2 * RING]
    gat = scratch[2 * RING:3 * RING]
    ob = scratch[3 * RING:4 * RING]
    pe_v = scratch[4 * RING]
    semg = scratch[4 * RING + 1:4 * RING + 1 + RING]
    sems = scratch[4 * RING + 1 + RING:4 * RING + 1 + 2 * RING]
    semi = scratch[4 * RING + 1 + 2 * RING:4 * RING + 1 + 3 * RING]
    _emb_body(B, x_hbm, tbl_hbm, pe_hbm, out_hbm,
              idxr, idxh, gat, ob, pe_v, semg, sems, semi)


def kernel(x, table, pe):
    B, seq = x.shape
    x_t = x.T.astype(jnp.int32)
    tbl2 = table.reshape(table.shape[0] // 2, 2 * D)
    pe_lin = pe[:seq].reshape(seq * D)

    scratch = (
        [pltpu.VMEM((NB,), jnp.int32) for _ in range(RING)]
        + [pltpu.VMEM((NB,), jnp.int32) for _ in range(RING)]
        + [pltpu.VMEM((NB, 2 * D), jnp.float32) for _ in range(RING)]
        + [pltpu.VMEM((D, NB), jnp.float32) for _ in range(RING)]
        + [pltpu.VMEM((S * D,), jnp.float32)]
        + [pltpu.SemaphoreType.DMA for _ in range(3 * RING)]
    )
    mesh = plsc.VectorSubcoreMesh(core_axis_name="c", subcore_axis_name="s")
    run = pl.kernel(
        functools.partial(_body_adapter, B),
        out_type=jax.ShapeDtypeStruct((seq, D, B), jnp.float32),
        mesh=mesh,
        scratch_types=scratch,
        compiler_params=pltpu.CompilerParams(
            use_tc_tiling_on_sc=True, needs_layout_passes=False),
    )
    out_t = run(x_t, tbl2, pe_lin)
    return out_t.transpose(2, 0, 1)


# speedup vs baseline: 2.4077x; 1.5304x over previous
import functools
import math

import jax
import jax.numpy as jnp
from jax import lax
from jax.experimental import pallas as pl
from jax.experimental.pallas import tpu as pltpu
from jax.experimental.pallas import tpu_sc as plsc

D = 64
S = 200
LANES = 16
NC, NS = 2, 16
NW = NC * NS
NB = 128
NG = NB // LANES
RING = 4

SCALE = math.sqrt(D)


def _emb_body(B, x_hbm, tbl_hbm, pe_hbm, out_hbm,
              idxr, idxh, gat, ob, pe_v, semg, sems, semi):
    wid = lax.axis_index("s") * NC + lax.axis_index("c")
    b0 = wid * NB

    lane = lax.iota(jnp.int32, LANES)

    def prep_and_fire(q):
        for g in range(NG):
            v = idxr[q][pl.ds(g * LANES, LANES)]
            idxh[q][pl.ds(g * LANES, LANES)] = lax.shift_right_logical(v, 1)
        pltpu.async_copy(tbl_hbm.at[idxh[q]], gat[q].at[:, pl.ds(0, 2 * D)],
                         semg[q])

    def compute(p, s):
        rows = []
        hoffs = []
        for g in range(NG):
            v = idxr[p][pl.ds(g * LANES, LANES)]
            rows.append(lane + (g * LANES))
            hoffs.append(lax.shift_left(lax.bitwise_and(v, 1), 6))

        def tbody(i):
            d0 = (i // LANES) * LANES
            t = i % LANES
            rot = lax.bitwise_and(lane + t, LANES - 1)
            dvec = rot + d0
            pv = plsc.load_gather(pe_v, [dvec + (s * D)])
            for g in range(NG):
                col = hoffs[g] + dvec
                val = plsc.load_gather(gat[p], [rows[g], col])
                plsc.store_scatter(ob[p], [dvec, rows[g]], val * SCALE + pv)

        plsc.parallel_loop(0, D, 1, unroll=8)(tbody)

    pltpu.sync_copy(pe_hbm, pe_v)

    pltpu.sync_copy(x_hbm.at[0, pl.ds(b0, NB)], idxr[0])
    prep_and_fire(0)
    pltpu.sync_copy(x_hbm.at[1, pl.ds(b0, NB)], idxr[1])
    prep_and_fire(1)
    pltpu.async_copy(x_hbm.at[2, pl.ds(b0, NB)], idxr[2], semi[2])

    def body_for(b, s):
        q = (b + 2) % RING
        r = (b + 3) % RING

        @pl.when(s + 2 < S)
        def _():
            @pl.when(s >= 2)
            def _():
                pltpu.make_async_copy(
                    ob[q], out_hbm.at[s - 2, :, pl.ds(b0, NB)],
                    sems[q]).wait()
            pltpu.make_async_copy(
                x_hbm.at[s + 2, pl.ds(b0, NB)], idxr[q], semi[q]).wait()
            prep_and_fire(q)

        pltpu.make_async_copy(
            tbl_hbm.at[pl.ds(0, NB)], gat[b].at[:, pl.ds(0, 2 * D)],
            semg[b]).wait()

        compute(b, s)

        @pl.when(s + 3 < S)
        def _():
            pltpu.async_copy(
                x_hbm.at[s + 3, pl.ds(b0, NB)], idxr[r], semi[r])

        pltpu.async_copy(ob[b], out_hbm.at[s, :, pl.ds(b0, NB)], sems[b])

    def outer(i, carry):
        for b in range(RING):
            body_for(b, i * RING + b)
        return carry

    lax.fori_loop(0, S // RING, outer, 0)

    for sf in range(S - RING, S):
        pltpu.make_async_copy(
            ob[sf % RING], out_hbm.at[sf, :, pl.ds(b0, NB)],
            sems[sf % RING]).wait()


def _body_adapter(B, x_hbm, tbl_hbm, pe_hbm, out_hbm, *scratch):
    idxr = scratch[0:RING]
    idxh = scratch[RING:2 * RING]
    gat = scratch[2 * RING:3 * RING]
    ob = scratch[3 * RING:4 * RING]
    pe_v = scratch[4 * RING]
    semg = scratch[4 * RING + 1:4 * RING + 1 + RING]
    sems = scratch[4 * RING + 1 + RING:4 * RING + 1 + 2 * RING]
    semi = scratch[4 * RING + 1 + 2 * RING:4 * RING + 1 + 3 * RING]
    _emb_body(B, x_hbm, tbl_hbm, pe_hbm, out_hbm,
              idxr, idxh, gat, ob, pe_v, semg, sems, semi)


def kernel(x, table, pe):
    B, seq = x.shape
    x_t = x.T.astype(jnp.int32)
    tbl2 = table.reshape(table.shape[0] // 2, 2 * D)
    pe_lin = pe[:seq].reshape(seq * D)

    scratch = (
        [pltpu.VMEM((NB,), jnp.int32) for _ in range(RING)]
        + [pltpu.VMEM((NB,), jnp.int32) for _ in range(RING)]
        + [pltpu.VMEM((NB, 2 * D), jnp.float32) for _ in range(RING)]
        + [pltpu.VMEM((D, NB), jnp.float32) for _ in range(RING)]
        + [pltpu.VMEM((S * D,), jnp.float32)]
        + [pltpu.SemaphoreType.DMA for _ in range(3 * RING)]
    )
    mesh = plsc.VectorSubcoreMesh(core_axis_name="c", subcore_axis_name="s")
    run = pl.kernel(
        functools.partial(_body_adapter, B),
        out_type=jax.ShapeDtypeStruct((seq, D, B), jnp.float32),
        mesh=mesh,
        scratch_types=scratch,
        compiler_params=pltpu.CompilerParams(
            use_tc_tiling_on_sc=True, needs_layout_passes=False),
    )
    out_t = run(x_t, tbl2, pe_lin)
    return out_t.transpose(2, 0, 1)
